# baseline (speedup 1.0000x reference)
import functools

import jax
import jax.numpy as jnp
from jax import lax
from jax.experimental import pallas as pl
from jax.experimental.pallas import tpu as pltpu
from jax.experimental.pallas import tpu_sc as plsc

NUM_CLASSES = 1000000
HIDDEN = 64
BATCH = 16384

_INFO = plsc.get_sparse_core_info()
_NC = _INFO.num_cores
_NS = _INFO.num_subcores
_NW = _NC * _NS
_BPW = BATCH // _NW
_CHUNK = 128
_NCHUNK = _BPW // _CHUNK

_mesh = plsc.VectorSubcoreMesh(core_axis_name="c", subcore_axis_name="s")


@functools.partial(
    pl.kernel,
    mesh=_mesh,
    out_type=jax.ShapeDtypeStruct((_NW, _NCHUNK, _CHUNK, HIDDEN), jnp.float32),
    scratch_types=[
        pltpu.VMEM((_NCHUNK, _CHUNK), jnp.int32),
        pltpu.VMEM((_NCHUNK, _CHUNK, HIDDEN), jnp.float32),
        pltpu.SemaphoreType.DMA,
    ],
    compiler_params=pltpu.CompilerParams(use_tc_tiling_on_sc=False),
)
def _gather_kernel(labels_hbm, table_hbm, out_hbm, idx_v, rows_v, sem):
    wid = lax.axis_index("s") * _NC + lax.axis_index("c")
    pltpu.sync_copy(labels_hbm.at[wid], idx_v)
    copies = [
        pltpu.async_copy(table_hbm.at[idx_v.at[j]], rows_v.at[j], sem)
        for j in range(_NCHUNK)
    ]
    for c in copies:
        c.wait()
    pltpu.sync_copy(rows_v, out_hbm.at[wid])


def kernel(labels, table):
    labels_r = labels.astype(jnp.int32).reshape(_NW, _NCHUNK, _CHUNK)
    out = _gather_kernel(labels_r, table)
    return out.reshape(BATCH, HIDDEN)


# speedup vs baseline: 1.0342x; 1.0342x over previous
import functools

import jax
import jax.numpy as jnp
from jax import lax
from jax.experimental import pallas as pl
from jax.experimental.pallas import tpu as pltpu
from jax.experimental.pallas import tpu_sc as plsc

NUM_CLASSES = 1000000
HIDDEN = 64
BATCH = 16384

_INFO = plsc.get_sparse_core_info()
_NC = _INFO.num_cores
_NS = _INFO.num_subcores
_NW = _NC * _NS
_BPW = BATCH // _NW

_mesh = plsc.VectorSubcoreMesh(core_axis_name="c", subcore_axis_name="s")


@functools.partial(
    pl.kernel,
    mesh=_mesh,
    out_type=jax.ShapeDtypeStruct((BATCH, HIDDEN), jnp.float32),
    scratch_types=[
        pltpu.VMEM((_BPW,), jnp.int32),
        pltpu.SemaphoreType.DMA,
    ],
)
def _gather_kernel(labels_hbm, table_hbm, out_hbm, idx_v, sem):
    wid = lax.axis_index("s") * _NC + lax.axis_index("c")
    base = wid * _BPW
    pltpu.sync_copy(labels_hbm.at[pl.ds(base, _BPW)], idx_v)

    def issue(g, _):
        vec = idx_v[pl.ds(g * 16, 16)]
        for k in range(16):
            pltpu.make_async_copy(
                table_hbm.at[pl.ds(vec[k], 1)],
                out_hbm.at[pl.ds(base + g * 16 + k, 1)],
                sem,
            ).start()
        return ()

    def drain(i, _):
        pltpu.make_async_copy(
            table_hbm.at[pl.ds(0, 1)], out_hbm.at[pl.ds(base + i, 1)], sem
        ).wait()
        return ()

    lax.fori_loop(0, _BPW // 16, issue, (), unroll=False)
    lax.fori_loop(0, _BPW, drain, (), unroll=False)


def kernel(labels, table):
    return _gather_kernel(labels.astype(jnp.int32), table)


# speedup vs baseline: 1.8285x; 1.7681x over previous
import functools

import jax
import jax.numpy as jnp
from jax import lax
from jax.experimental import pallas as pl
from jax.experimental.pallas import tpu as pltpu
from jax.experimental.pallas import tpu_sc as plsc

NUM_CLASSES = 1000000
HIDDEN = 64
BATCH = 16384

_INFO = plsc.get_sparse_core_info()
_NC = _INFO.num_cores
_NS = _INFO.num_subcores
_NW = _NC * _NS
_LANES = _INFO.num_lanes

_TILE = 128
_NTC = (NUM_CLASSES + 1) // _TILE
_SCAN_CLASSES = _NTC * _TILE
_TAIL = NUM_CLASSES + 1 - _SCAN_CLASSES

_TC_PER_W = _NTC // _NW
_TC_EXTRA = _NTC - _TC_PER_W * _NW
_WIN = 512
_N_WIN = _TC_PER_W * _TILE // _WIN

_FLUSH_CAP = 128
_FLUSH_HI = _FLUSH_CAP - _LANES
_STAGE_ROWS = BATCH + _FLUSH_CAP

_mesh = plsc.VectorSubcoreMesh(core_axis_name="c", subcore_axis_name="s")


def _popcount(mask):
    return plsc.all_reduce_population_count(mask)[0]


@functools.partial(
    pl.kernel,
    mesh=_mesh,
    out_type=jax.ShapeDtypeStruct((_STAGE_ROWS, _TILE), jnp.float32),
    scratch_types=[
        pltpu.VMEM((BATCH,), jnp.int32),
        pltpu.VMEM((BATCH,), jnp.int32),
        pltpu.VMEM((HIDDEN, _WIN), jnp.float32),
        pltpu.VMEM((_FLUSH_CAP, _TILE), jnp.float32),
        pltpu.VMEM((_FLUSH_CAP,), jnp.int32),
        pltpu.SemaphoreType.DMA,
    ],
    compiler_params=pltpu.CompilerParams(
        use_tc_tiling_on_sc=True, needs_layout_passes=False
    ),
)
def _scan_kernel(labels_hbm, table_t_hbm, stage_hbm,
                 labels_v, pos_v, win_v, rows_v, rpos_v, sem):
    wid = lax.axis_index("s") * _NC + lax.axis_index("c")
    extra = jnp.minimum(wid, _TC_EXTRA)
    lo = (wid * _TC_PER_W + extra) * _TILE
    n_win = _N_WIN + jnp.where(wid < _TC_EXTRA, 1, 0)
    hi = lo + _TC_PER_W * _TILE + jnp.where(wid < _TC_EXTRA, _TILE, 0)

    pltpu.sync_copy(labels_hbm, labels_v)
    lane = lax.iota(jnp.int32, _LANES)

    def compact(i, off):
        vec = labels_v[pl.ds(i * _LANES, _LANES)]
        m = (vec >= lo) & (vec < hi)
        plsc.store_compressed(pos_v.at[pl.ds(off, _LANES)], lane + i * _LANES, mask=m)
        return off + _popcount(m)

    mcount = lax.fori_loop(0, BATCH // _LANES, compact, 0, unroll=False)
    mvregs = (mcount + _LANES - 1) // _LANES

    def init_rpos(k, _):
        rpos_v[pl.ds(k * _LANES, _LANES)] = lane + (BATCH + k * _LANES)
        return ()

    lax.fori_loop(0, _FLUSH_CAP // _LANES, init_rpos, (), unroll=False)

    def flush(fcnt):
        copy = pltpu.make_async_copy(rows_v, stage_hbm.at[rpos_v], sem)
        copy.start()
        copy.wait()
        lax.fori_loop(0, _FLUSH_CAP // _LANES, init_rpos, (), unroll=False)
        del fcnt
        return 0

    def window(w, fcnt):
        woff = lo + w * _WIN
        woff = jnp.minimum(woff, _SCAN_CLASSES - _WIN)
        pltpu.sync_copy(table_t_hbm.at[:, pl.ds(woff, _WIN)], win_v)

        def scan_matches(j, fcnt):
            valid = (lane + j * _LANES) < mcount
            vpos = pos_v[pl.ds(j * _LANES, _LANES)]
            vcls = plsc.load_gather(labels_v, [vpos], mask=valid)
            inw = valid & (vcls >= woff) & (vcls < woff + _WIN) & (vcls < hi)
            cnt = _popcount(inw)

            def extract(fcnt):
                rel = jnp.where(inw, vcls - woff, 0)
                mi32 = inw.astype(jnp.int32)
                dest = fcnt + lax.cumsum(mi32) - mi32
                for f in range(HIDDEN):
                    vals = plsc.load_gather(
                        win_v, [jnp.full((_LANES,), f, jnp.int32), rel],
                        mask=inw)
                    plsc.store_scatter(
                        rows_v, [dest, jnp.full((_LANES,), f, jnp.int32)],
                        vals, mask=inw)
                plsc.store_scatter(rpos_v, [dest], vpos, mask=inw)
                return fcnt + cnt

            fcnt = lax.cond(cnt > 0, extract, lambda c: c, fcnt)
            fcnt = lax.cond(fcnt >= _FLUSH_HI, flush, lambda c: c, fcnt)
            return fcnt

        return lax.fori_loop(0, mvregs, scan_matches, fcnt, unroll=False)

    fcnt = lax.fori_loop(0, n_win, window, 0, unroll=False)
    lax.cond(fcnt > 0, flush, lambda c: c, fcnt)


def kernel(labels, table):
    labels = labels.astype(jnp.int32)
    stage = _scan_kernel(labels, table.T)
    out = stage[:BATCH, :HIDDEN]
    tail_table = table[_SCAN_CLASSES:]
    tail_idx = jnp.clip(labels - _SCAN_CLASSES, 0, _TAIL - 1)
    tail_rows = jnp.take(tail_table, tail_idx, axis=0)
    return jnp.where((labels >= _SCAN_CLASSES)[:, None], tail_rows, out)


# speedup vs baseline: 2.6639x; 1.4568x over previous
import functools

import jax
import jax.numpy as jnp
from jax import lax
from jax.experimental import pallas as pl
from jax.experimental.pallas import tpu as pltpu
from jax.experimental.pallas import tpu_sc as plsc

NUM_CLASSES = 1000000
HIDDEN = 64
BATCH = 16384

_INFO = plsc.get_sparse_core_info()
_NC = _INFO.num_cores
_NS = _INFO.num_subcores
_NW = _NC * _NS
_LANES = _INFO.num_lanes

_TILE = 128
_NTC = (NUM_CLASSES + 1) // _TILE
_SCAN_CLASSES = _NTC * _TILE
_TAIL = NUM_CLASSES + 1 - _SCAN_CLASSES

_TC_PER_W = _NTC // _NW
_TC_EXTRA = _NTC - _TC_PER_W * _NW
_WIN = 512
_N_WIN = _TC_PER_W * _TILE // _WIN

_FLUSH_CAP = 128
_FLUSH_HI = _FLUSH_CAP - _LANES
_STAGE_ROWS = BATCH + _FLUSH_CAP

_mesh = plsc.VectorSubcoreMesh(core_axis_name="c", subcore_axis_name="s")


def _popcount(mask):
    return plsc.all_reduce_population_count(mask)[0]


@functools.partial(
    pl.kernel,
    mesh=_mesh,
    out_type=jax.ShapeDtypeStruct((_STAGE_ROWS, _TILE), jnp.float32),
    scratch_types=[
        pltpu.VMEM((BATCH,), jnp.int32),
        pltpu.VMEM((BATCH,), jnp.int32),
        pltpu.VMEM((HIDDEN, _WIN), jnp.float32),
        pltpu.VMEM((HIDDEN, _WIN), jnp.float32),
        pltpu.VMEM((_FLUSH_CAP, _TILE), jnp.float32),
        pltpu.VMEM((_FLUSH_CAP,), jnp.int32),
        pltpu.SemaphoreType.DMA,
        pltpu.SemaphoreType.DMA,
        pltpu.SemaphoreType.DMA,
    ],
    compiler_params=pltpu.CompilerParams(
        use_tc_tiling_on_sc=True, needs_layout_passes=False
    ),
)
def _scan_kernel(labels_hbm, table_t_hbm, stage_hbm,
                 labels_v, pos_v, win_a, win_b, rows_v, rpos_v,
                 sem_a, sem_b, sem):
    wid = lax.axis_index("s") * _NC + lax.axis_index("c")
    extra = jnp.minimum(wid, _TC_EXTRA)
    lo = (wid * _TC_PER_W + extra) * _TILE
    n_win = _N_WIN + jnp.where(wid < _TC_EXTRA, 1, 0)
    hi = lo + _TC_PER_W * _TILE + jnp.where(wid < _TC_EXTRA, _TILE, 0)

    pltpu.sync_copy(labels_hbm, labels_v)
    lane = lax.iota(jnp.int32, _LANES)

    def compact(i, off):
        vec = labels_v[pl.ds(i * _LANES, _LANES)]
        m = (vec >= lo) & (vec < hi)
        plsc.store_compressed(pos_v.at[pl.ds(off, _LANES)], lane + i * _LANES, mask=m)
        return off + _popcount(m)

    mcount = lax.fori_loop(0, BATCH // _LANES, compact, 0, unroll=False)
    mvregs = (mcount + _LANES - 1) // _LANES

    def init_rpos(k, _):
        rpos_v[pl.ds(k * _LANES, _LANES)] = lane + (BATCH + k * _LANES)
        return ()

    lax.fori_loop(0, _FLUSH_CAP // _LANES, init_rpos, (), unroll=False)

    def flush(fcnt):
        copy = pltpu.make_async_copy(rows_v, stage_hbm.at[rpos_v], sem)
        copy.start()
        copy.wait()
        lax.fori_loop(0, _FLUSH_CAP // _LANES, init_rpos, (), unroll=False)
        del fcnt
        return 0

    def win_off(w):
        return jnp.minimum(lo + w * _WIN, _SCAN_CLASSES - _WIN)

    def start(w, win_v, sem_w):
        pltpu.make_async_copy(
            table_t_hbm.at[:, pl.ds(win_off(w), _WIN)], win_v, sem_w
        ).start()

    def wait(win_v, sem_w):
        pltpu.make_async_copy(
            table_t_hbm.at[:, pl.ds(0, _WIN)], win_v, sem_w
        ).wait()

    def extract_window(w, win_v, fcnt):
        woff = win_off(w)

        def scan_matches(j, fcnt):
            valid = (lane + j * _LANES) < mcount
            vpos = pos_v[pl.ds(j * _LANES, _LANES)]
            vcls = plsc.load_gather(labels_v, [vpos], mask=valid)
            inw = valid & (vcls >= woff) & (vcls < woff + _WIN) & (vcls < hi)
            cnt = _popcount(inw)

            def extract(fcnt):
                rel = jnp.where(inw, vcls - woff, 0)
                mi32 = inw.astype(jnp.int32)
                dest = fcnt + lax.cumsum(mi32) - mi32
                for f in range(HIDDEN):
                    vals = plsc.load_gather(
                        win_v, [jnp.full((_LANES,), f, jnp.int32), rel],
                        mask=inw)
                    plsc.store_scatter(
                        rows_v, [dest, jnp.full((_LANES,), f, jnp.int32)],
                        vals, mask=inw)
                plsc.store_scatter(rpos_v, [dest], vpos, mask=inw)
                return fcnt + cnt

            fcnt = lax.cond(cnt > 0, extract, lambda c: c, fcnt)
            fcnt = lax.cond(fcnt >= _FLUSH_HI, flush, lambda c: c, fcnt)
            return fcnt

        return lax.fori_loop(0, mvregs, scan_matches, fcnt, unroll=False)

    start(0, win_a, sem_a)

    def pair(w2, fcnt):
        w0 = 2 * w2
        w1 = w0 + 1
        wait(win_a, sem_a)

        def with_b(fcnt):
            start(w1, win_b, sem_b)
            fcnt = extract_window(w0, win_a, fcnt)
            wait(win_b, sem_b)
            fcnt = lax.cond(
                w1 + 1 < n_win,
                lambda c: (start(w1 + 1, win_a, sem_a), c)[1],
                lambda c: c,
                fcnt,
            )
            return extract_window(w1, win_b, fcnt)

        return lax.cond(
            w1 < n_win, with_b, lambda c: extract_window(w0, win_a, c), fcnt
        )

    fcnt = lax.fori_loop(0, (_N_WIN + 2) // 2, pair, 0, unroll=False)
    lax.cond(fcnt > 0, flush, lambda c: c, fcnt)


def kernel(labels, table):
    labels = labels.astype(jnp.int32)
    stage = _scan_kernel(labels, table.T)
    out = stage[:BATCH, :HIDDEN]
    tail_table = table[_SCAN_CLASSES:]
    tail_idx = jnp.clip(labels - _SCAN_CLASSES, 0, _TAIL - 1)
    tail_rows = jnp.take(tail_table, tail_idx, axis=0)
    return jnp.where((labels >= _SCAN_CLASSES)[:, None], tail_rows, out)


# speedup vs baseline: 3.5081x; 1.3169x over previous
import functools

import jax
import jax.numpy as jnp
from jax import lax
from jax.experimental import pallas as pl
from jax.experimental.pallas import tpu as pltpu
from jax.experimental.pallas import tpu_sc as plsc

NUM_CLASSES = 1000000
HIDDEN = 64
BATCH = 16384

_INFO = plsc.get_sparse_core_info()
_NC = _INFO.num_cores
_NS = _INFO.num_subcores
_NW = _NC * _NS
_LANES = _INFO.num_lanes

_TILE = 128
_NTC = (NUM_CLASSES + 1) // _TILE
_SCAN_CLASSES = _NTC * _TILE
_TAIL = NUM_CLASSES + 1 - _SCAN_CLASSES

_TC_PER_W = _NTC // _NW
_TC_EXTRA = _NTC - _TC_PER_W * _NW
_WIN = 512
_N_WIN = _TC_PER_W * _TILE // _WIN

_FLUSH_CAP = 128
_FLUSH_HI = _FLUSH_CAP - _LANES
_BLK = 128
_STAGE_ROWS = BATCH + _FLUSH_CAP

_mesh = plsc.VectorSubcoreMesh(core_axis_name="c", subcore_axis_name="s")


def _popcount(mask):
    return plsc.all_reduce_population_count(mask)[0]


@functools.partial(
    pl.kernel,
    mesh=_mesh,
    out_type=jax.ShapeDtypeStruct((_STAGE_ROWS, _TILE), jnp.float32),
    scratch_types=[
        pltpu.VMEM((BATCH,), jnp.int32),
        pltpu.VMEM((BATCH,), jnp.int32),
        pltpu.VMEM((HIDDEN, _WIN), jnp.float32),
        pltpu.VMEM((HIDDEN, _WIN), jnp.float32),
        pltpu.VMEM((_FLUSH_CAP, _TILE), jnp.float32),
        pltpu.VMEM((_FLUSH_CAP,), jnp.int32),
        pltpu.VMEM((_BLK * _LANES,), jnp.int32),
        pltpu.VMEM((_BLK * _LANES,), jnp.int32),
        pltpu.SemaphoreType.DMA,
        pltpu.SemaphoreType.DMA,
        pltpu.SemaphoreType.DMA,
    ],
    compiler_params=pltpu.CompilerParams(
        use_tc_tiling_on_sc=True, needs_layout_passes=False
    ),
)
def _scan_kernel(labels_hbm, table_t_hbm, stage_hbm,
                 labels_v, pos_v, win_a, win_b, rows_v, rpos_v,
                 wrel_v, wpos_v, sem_a, sem_b, sem):
    wid = lax.axis_index("s") * _NC + lax.axis_index("c")
    extra = jnp.minimum(wid, _TC_EXTRA)
    lo = (wid * _TC_PER_W + extra) * _TILE
    n_win = _N_WIN + jnp.where(wid < _TC_EXTRA, 1, 0)
    hi = lo + _TC_PER_W * _TILE + jnp.where(wid < _TC_EXTRA, _TILE, 0)

    pltpu.sync_copy(labels_hbm, labels_v)
    lane = lax.iota(jnp.int32, _LANES)

    def compact(i, off):
        vec = labels_v[pl.ds(i * _LANES, _LANES)]
        m = (vec >= lo) & (vec < hi)
        plsc.store_compressed(pos_v.at[pl.ds(off, _LANES)], lane + i * _LANES, mask=m)
        return off + _popcount(m)

    mcount = lax.fori_loop(0, BATCH // _LANES, compact, 0, unroll=False)
    mvregs = (mcount + _LANES - 1) // _LANES

    def init_rpos(k, _):
        rpos_v[pl.ds(k * _LANES, _LANES)] = lane + (BATCH + k * _LANES)
        return ()

    lax.fori_loop(0, _FLUSH_CAP // _LANES, init_rpos, (), unroll=False)

    def flush(fcnt):
        copy = pltpu.make_async_copy(rows_v, stage_hbm.at[rpos_v], sem)
        copy.start()
        copy.wait()
        lax.fori_loop(0, _FLUSH_CAP // _LANES, init_rpos, (), unroll=False)
        del fcnt
        return 0

    def win_off(w):
        return jnp.minimum(lo + w * _WIN, _SCAN_CLASSES - _WIN)

    def start(w, win_v, sem_w):
        pltpu.make_async_copy(
            table_t_hbm.at[:, pl.ds(win_off(w), _WIN)], win_v, sem_w
        ).start()

    def wait(win_v, sem_w):
        pltpu.make_async_copy(
            table_t_hbm.at[:, pl.ds(0, _WIN)], win_v, sem_w
        ).wait()

    def extract_window(w, win_v, fcnt):
        woff = win_off(w)

        def compress(j, wcnt):
            valid = (lane + j * _LANES) < mcount
            vpos = pos_v[pl.ds(j * _LANES, _LANES)]
            vcls = plsc.load_gather(labels_v, [vpos], mask=valid)
            inw = valid & (vcls >= woff) & (vcls < woff + _WIN) & (vcls < hi)
            plsc.store_compressed(
                wrel_v.at[pl.ds(wcnt, _LANES)], vcls - woff, mask=inw)
            plsc.store_compressed(
                wpos_v.at[pl.ds(wcnt, _LANES)], vpos, mask=inw)
            return wcnt + _popcount(inw)

        def dense(q, fcnt):
            valid = (lane + q * _LANES) < fcnt[1]
            rel = jnp.where(valid, wrel_v[pl.ds(q * _LANES, _LANES)], 0)
            vpos = wpos_v[pl.ds(q * _LANES, _LANES)]
            dest = fcnt[0] + lane
            for f in range(HIDDEN):
                vals = plsc.load_gather(
                    win_v, [jnp.full((_LANES,), f, jnp.int32), rel],
                    mask=valid)
                plsc.store_scatter(
                    rows_v, [dest, jnp.full((_LANES,), f, jnp.int32)],
                    vals, mask=valid)
            plsc.store_scatter(rpos_v, [dest], vpos, mask=valid)
            new = fcnt[0] + _popcount(valid)
            new = lax.cond(new >= _FLUSH_HI, flush, lambda c: c, new)
            return (new, fcnt[1])

        def block(b, fcnt):
            j0 = b * _BLK
            jn = jnp.minimum(mvregs - j0, _BLK)
            wcnt = lax.fori_loop(
                0, jn, lambda j, wc: compress(j0 + j, wc), 0, unroll=False)
            fcnt, _ = lax.fori_loop(
                0, (wcnt + _LANES - 1) // _LANES, dense, (fcnt, wcnt),
                unroll=False)
            return fcnt

        nblk = (mvregs + _BLK - 1) // _BLK
        return lax.fori_loop(0, nblk, block, fcnt, unroll=False)

    start(0, win_a, sem_a)

    def pair(w2, fcnt):
        w0 = 2 * w2
        w1 = w0 + 1
        wait(win_a, sem_a)

        def with_b(fcnt):
            start(w1, win_b, sem_b)
            fcnt = extract_window(w0, win_a, fcnt)
            wait(win_b, sem_b)
            fcnt = lax.cond(
                w1 + 1 < n_win,
                lambda c: (start(w1 + 1, win_a, sem_a), c)[1],
                lambda c: c,
                fcnt,
            )
            return extract_window(w1, win_b, fcnt)

        return lax.cond(
            w1 < n_win, with_b, lambda c: extract_window(w0, win_a, c), fcnt
        )

    fcnt = lax.fori_loop(0, (_N_WIN + 2) // 2, pair, 0, unroll=False)
    lax.cond(fcnt > 0, flush, lambda c: c, fcnt)


def kernel(labels, table):
    labels = labels.astype(jnp.int32)
    stage = _scan_kernel(labels, table.T)
    out = stage[:BATCH, :HIDDEN]
    tail_table = table[_SCAN_CLASSES:]
    tail_idx = jnp.clip(labels - _SCAN_CLASSES, 0, _TAIL - 1)
    tail_rows = jnp.take(tail_table, tail_idx, axis=0)
    return jnp.where((labels >= _SCAN_CLASSES)[:, None], tail_rows, out)


# speedup vs baseline: 3.7826x; 1.0783x over previous
import functools

import jax
import jax.numpy as jnp
from jax import lax
from jax.experimental import pallas as pl
from jax.experimental.pallas import tpu as pltpu
from jax.experimental.pallas import tpu_sc as plsc

NUM_CLASSES = 1000000
HIDDEN = 64
BATCH = 16384

_INFO = plsc.get_sparse_core_info()
_NC = _INFO.num_cores
_NS = _INFO.num_subcores
_NW = _NC * _NS
_LANES = _INFO.num_lanes

_TILE = 128
_NTC = (NUM_CLASSES + 1) // _TILE
_SCAN_CLASSES = _NTC * _TILE
_TAIL = NUM_CLASSES + 1 - _SCAN_CLASSES

_TC_PER_W = _NTC // _NW
_TC_EXTRA = _NTC - _TC_PER_W * _NW
_WIN = 512
_N_WIN = _TC_PER_W * _TILE // _WIN

_FLUSH_CAP = 128
_FLUSH_HI = _FLUSH_CAP - _LANES
_BLK = 128
_STAGE_ROWS = BATCH + _FLUSH_CAP

_mesh = plsc.VectorSubcoreMesh(core_axis_name="c", subcore_axis_name="s")


def _popcount(mask):
    return plsc.all_reduce_population_count(mask)[0]


@functools.partial(
    pl.kernel,
    mesh=_mesh,
    out_type=jax.ShapeDtypeStruct((_STAGE_ROWS, _TILE), jnp.float32),
    scratch_types=[
        pltpu.VMEM((BATCH,), jnp.int32),
        pltpu.VMEM((BATCH,), jnp.int32),
        pltpu.VMEM((HIDDEN, _WIN), jnp.float32),
        pltpu.VMEM((HIDDEN, _WIN), jnp.float32),
        pltpu.VMEM((_FLUSH_CAP, _TILE), jnp.float32),
        pltpu.VMEM((_FLUSH_CAP,), jnp.int32),
        pltpu.VMEM((_BLK * _LANES,), jnp.int32),
        pltpu.VMEM((_BLK * _LANES,), jnp.int32),
        pltpu.SemaphoreType.DMA,
        pltpu.SemaphoreType.DMA,
        pltpu.SemaphoreType.DMA,
    ],
    compiler_params=pltpu.CompilerParams(
        use_tc_tiling_on_sc=True, needs_layout_passes=False
    ),
)
def _scan_kernel(labels_hbm, table_t_hbm, stage_hbm,
                 labels_v, pos_v, win_a, win_b, rows_v, rpos_v,
                 wrel_v, wpos_v, sem_a, sem_b, sem):
    wid = lax.axis_index("s") * _NC + lax.axis_index("c")
    extra = jnp.minimum(wid, _TC_EXTRA)
    lo = (wid * _TC_PER_W + extra) * _TILE
    n_win = _N_WIN + jnp.where(wid < _TC_EXTRA, 1, 0)
    hi = lo + _TC_PER_W * _TILE + jnp.where(wid < _TC_EXTRA, _TILE, 0)

    pltpu.sync_copy(labels_hbm, labels_v)
    lane = lax.iota(jnp.int32, _LANES)

    def compact(i, off):
        vec = labels_v[pl.ds(i * _LANES, _LANES)]
        m = (vec >= lo) & (vec < hi)
        plsc.store_compressed(pos_v.at[pl.ds(off, _LANES)], lane + i * _LANES, mask=m)
        return off + _popcount(m)

    mcount = lax.fori_loop(0, BATCH // _LANES, compact, 0, unroll=False)
    mvregs = (mcount + _LANES - 1) // _LANES

    def init_rpos(k, _):
        rpos_v[pl.ds(k * _LANES, _LANES)] = lane + (BATCH + k * _LANES)
        return ()

    lax.fori_loop(0, _FLUSH_CAP // _LANES, init_rpos, (), unroll=False)

    def flush(fcnt):
        copy = pltpu.make_async_copy(rows_v, stage_hbm.at[rpos_v], sem)
        copy.start()
        copy.wait()
        lax.fori_loop(0, _FLUSH_CAP // _LANES, init_rpos, (), unroll=False)
        del fcnt
        return 0

    def win_off(w):
        return jnp.minimum(lo + w * _WIN, _SCAN_CLASSES - _WIN)

    def start(w, win_v, sem_w):
        pltpu.make_async_copy(
            table_t_hbm.at[:, pl.ds(win_off(w), _WIN)], win_v, sem_w
        ).start()

    def wait(win_v, sem_w):
        pltpu.make_async_copy(
            table_t_hbm.at[:, pl.ds(0, _WIN)], win_v, sem_w
        ).wait()

    def extract_window(w, win_v, fcnt):
        woff = win_off(w)

        def compress(j, wcnt):
            valid = (lane + j * _LANES) < mcount
            vpos = pos_v[pl.ds(j * _LANES, _LANES)]
            vcls = plsc.load_gather(labels_v, [vpos], mask=valid)
            inw = valid & (vcls >= woff) & (vcls < woff + _WIN) & (vcls < hi)
            plsc.store_compressed(
                wrel_v.at[pl.ds(wcnt, _LANES)], vcls - woff, mask=inw)
            plsc.store_compressed(
                wpos_v.at[pl.ds(wcnt, _LANES)], vpos, mask=inw)
            return wcnt + _popcount(inw)

        def dense(q, fcnt):
            valid = (lane + q * _LANES) < fcnt[1]
            rel = jnp.where(valid, wrel_v[pl.ds(q * _LANES, _LANES)], 0)
            vpos = wpos_v[pl.ds(q * _LANES, _LANES)]
            dest = fcnt[0] + lane
            for f in range(HIDDEN):
                vals = plsc.load_gather(
                    win_v, [jnp.full((_LANES,), f, jnp.int32), rel],
                    mask=valid)
                plsc.store_scatter(
                    rows_v, [dest, jnp.full((_LANES,), f, jnp.int32)],
                    vals, mask=valid)
            plsc.store_scatter(rpos_v, [dest], vpos, mask=valid)
            new = fcnt[0] + _popcount(valid)
            new = lax.cond(new >= _FLUSH_HI, flush, lambda c: c, new)
            return (new, fcnt[1])

        def block(b, fcnt):
            j0 = b * _BLK
            jn = jnp.minimum(mvregs - j0, _BLK)
            wcnt = lax.fori_loop(
                0, jn, lambda j, wc: compress(j0 + j, wc), 0, unroll=False)
            fcnt, _ = lax.fori_loop(
                0, (wcnt + _LANES - 1) // _LANES, dense, (fcnt, wcnt),
                unroll=False)
            return fcnt

        nblk = (mvregs + _BLK - 1) // _BLK
        return lax.fori_loop(0, nblk, block, fcnt, unroll=False)

    start(0, win_a, sem_a)
    start(1, win_b, sem_b)

    def half(w, win_v, sem_w, fcnt):
        wait(win_v, sem_w)
        fcnt = extract_window(w, win_v, fcnt)
        return lax.cond(
            w + 2 < n_win,
            lambda c: (start(w + 2, win_v, sem_w), c)[1],
            lambda c: c,
            fcnt,
        )

    def pair(w2, fcnt):
        w0 = 2 * w2
        w1 = w0 + 1
        fcnt = half(w0, win_a, sem_a, fcnt)
        return lax.cond(
            w1 < n_win,
            lambda c: half(w1, win_b, sem_b, c),
            lambda c: c,
            fcnt,
        )

    fcnt = lax.fori_loop(0, (_N_WIN + 2) // 2, pair, 0, unroll=False)
    lax.cond(fcnt > 0, flush, lambda c: c, fcnt)


def kernel(labels, table):
    labels = labels.astype(jnp.int32)
    stage = _scan_kernel(labels, table.T)
    out = stage[:BATCH, :HIDDEN]
    tail_table = table[_SCAN_CLASSES:]
    tail_idx = jnp.clip(labels - _SCAN_CLASSES, 0, _TAIL - 1)
    tail_rows = jnp.take(tail_table, tail_idx, axis=0)
    return jnp.where((labels >= _SCAN_CLASSES)[:, None], tail_rows, out)


# speedup vs baseline: 3.8111x; 1.0075x over previous
import functools

import jax
import jax.numpy as jnp
from jax import lax
from jax.experimental import pallas as pl
from jax.experimental.pallas import tpu as pltpu
from jax.experimental.pallas import tpu_sc as plsc

NUM_CLASSES = 1000000
HIDDEN = 64
BATCH = 16384

_INFO = plsc.get_sparse_core_info()
_NC = _INFO.num_cores
_NS = _INFO.num_subcores
_NW = _NC * _NS
_LANES = _INFO.num_lanes

_TILE = 128
_NTC = (NUM_CLASSES + 1) // _TILE
_SCAN_CLASSES = _NTC * _TILE
_TAIL = NUM_CLASSES + 1 - _SCAN_CLASSES

_TC_PER_W = _NTC // _NW
_TC_EXTRA = _NTC - _TC_PER_W * _NW
_WIN = 512
_N_WIN = _TC_PER_W * _TILE // _WIN

_FLUSH_CAP = 128
_FLUSH_HI = _FLUSH_CAP - _LANES
_BLK = 128
_STAGE_ROWS = BATCH + _FLUSH_CAP

_mesh = plsc.VectorSubcoreMesh(core_axis_name="c", subcore_axis_name="s")


def _popcount(mask):
    return plsc.all_reduce_population_count(mask)[0]


@functools.partial(
    pl.kernel,
    mesh=_mesh,
    out_type=jax.ShapeDtypeStruct((_STAGE_ROWS, _TILE), jnp.float32),
    scratch_types=[
        pltpu.VMEM((BATCH,), jnp.int32),
        pltpu.VMEM((BATCH,), jnp.int32),
        pltpu.VMEM((HIDDEN, _WIN), jnp.float32),
        pltpu.VMEM((HIDDEN, _WIN), jnp.float32),
        pltpu.VMEM((_FLUSH_CAP, _TILE), jnp.float32),
        pltpu.VMEM((_FLUSH_CAP,), jnp.int32),
        pltpu.VMEM((_BLK * _LANES,), jnp.int32),
        pltpu.VMEM((_BLK * _LANES,), jnp.int32),
        pltpu.SemaphoreType.DMA,
        pltpu.SemaphoreType.DMA,
        pltpu.SemaphoreType.DMA,
    ],
    compiler_params=pltpu.CompilerParams(
        use_tc_tiling_on_sc=True, needs_layout_passes=False
    ),
)
def _scan_kernel(labels_hbm, table_t_hbm, stage_hbm,
                 labels_v, pos_v, win_a, win_b, rows_v, rpos_v,
                 wrel_v, wpos_v, sem_a, sem_b, sem):
    wid = lax.axis_index("s") * _NC + lax.axis_index("c")
    extra = jnp.minimum(wid, _TC_EXTRA)
    lo = (wid * _TC_PER_W + extra) * _TILE
    n_win = _N_WIN + jnp.where(wid < _TC_EXTRA, 1, 0)
    hi = lo + _TC_PER_W * _TILE + jnp.where(wid < _TC_EXTRA, _TILE, 0)

    lane = lax.iota(jnp.int32, _LANES)

    def compact(i, off):
        vec = labels_v[pl.ds(i * _LANES, _LANES)]
        m = (vec >= lo) & (vec < hi)
        plsc.store_compressed(pos_v.at[pl.ds(off, _LANES)], lane + i * _LANES, mask=m)
        return off + _popcount(m)

    def init_rpos(k, _):
        rpos_v[pl.ds(k * _LANES, _LANES)] = lane + (BATCH + k * _LANES)
        return ()

    lax.fori_loop(0, _FLUSH_CAP // _LANES, init_rpos, (), unroll=False)

    def flush(fcnt):
        copy = pltpu.make_async_copy(rows_v, stage_hbm.at[rpos_v], sem)
        copy.start()
        copy.wait()
        lax.fori_loop(0, _FLUSH_CAP // _LANES, init_rpos, (), unroll=False)
        del fcnt
        return 0

    def win_off(w):
        return jnp.minimum(lo + w * _WIN, _SCAN_CLASSES - _WIN)

    def start(w, win_v, sem_w):
        pltpu.make_async_copy(
            table_t_hbm.at[:, pl.ds(win_off(w), _WIN)], win_v, sem_w
        ).start()

    def wait(win_v, sem_w):
        pltpu.make_async_copy(
            table_t_hbm.at[:, pl.ds(0, _WIN)], win_v, sem_w
        ).wait()

    def extract_window(w, win_v, fcnt):
        woff = win_off(w)

        def compress(j, wcnt):
            valid = (lane + j * _LANES) < mcount
            vpos = pos_v[pl.ds(j * _LANES, _LANES)]
            vcls = plsc.load_gather(labels_v, [vpos], mask=valid)
            inw = valid & (vcls >= woff) & (vcls < woff + _WIN) & (vcls < hi)
            plsc.store_compressed(
                wrel_v.at[pl.ds(wcnt, _LANES)], vcls - woff, mask=inw)
            plsc.store_compressed(
                wpos_v.at[pl.ds(wcnt, _LANES)], vpos, mask=inw)
            return wcnt + _popcount(inw)

        def dense(q, fcnt):
            valid = (lane + q * _LANES) < fcnt[1]
            rel = jnp.where(valid, wrel_v[pl.ds(q * _LANES, _LANES)], 0)
            vpos = wpos_v[pl.ds(q * _LANES, _LANES)]
            dest = fcnt[0] + lane
            for f in range(HIDDEN):
                vals = plsc.load_gather(
                    win_v, [jnp.full((_LANES,), f, jnp.int32), rel],
                    mask=valid)
                plsc.store_scatter(
                    rows_v, [dest, jnp.full((_LANES,), f, jnp.int32)],
                    vals, mask=valid)
            plsc.store_scatter(rpos_v, [dest], vpos, mask=valid)
            new = fcnt[0] + _popcount(valid)
            new = lax.cond(new >= _FLUSH_HI, flush, lambda c: c, new)
            return (new, fcnt[1])

        def block(b, fcnt):
            j0 = b * _BLK
            jn = jnp.minimum(mvregs - j0, _BLK)
            wcnt = lax.fori_loop(
                0, jn, lambda j, wc: compress(j0 + j, wc), 0, unroll=False)
            fcnt, _ = lax.fori_loop(
                0, (wcnt + _LANES - 1) // _LANES, dense, (fcnt, wcnt),
                unroll=False)
            return fcnt

        nblk = (mvregs + _BLK - 1) // _BLK
        return lax.fori_loop(0, nblk, block, fcnt, unroll=False)

    start(0, win_a, sem_a)
    start(1, win_b, sem_b)
    labels_copy = pltpu.make_async_copy(labels_hbm, labels_v, sem)
    labels_copy.start()
    labels_copy.wait()
    mcount = lax.fori_loop(0, BATCH // _LANES, compact, 0, unroll=False)
    mvregs = (mcount + _LANES - 1) // _LANES

    def half(w, win_v, sem_w, fcnt):
        wait(win_v, sem_w)
        fcnt = extract_window(w, win_v, fcnt)
        return lax.cond(
            w + 2 < n_win,
            lambda c: (start(w + 2, win_v, sem_w), c)[1],
            lambda c: c,
            fcnt,
        )

    def pair(w2, fcnt):
        w0 = 2 * w2
        w1 = w0 + 1
        fcnt = half(w0, win_a, sem_a, fcnt)
        return lax.cond(
            w1 < n_win,
            lambda c: half(w1, win_b, sem_b, c),
            lambda c: c,
            fcnt,
        )

    fcnt = lax.fori_loop(0, (_N_WIN + 2) // 2, pair, 0, unroll=False)
    lax.cond(fcnt > 0, flush, lambda c: c, fcnt)


def kernel(labels, table):
    labels = labels.astype(jnp.int32)
    stage = _scan_kernel(labels, table.T)
    out = stage[:BATCH, :HIDDEN]
    tail_table = table[_SCAN_CLASSES:]
    tail_idx = jnp.clip(labels - _SCAN_CLASSES, 0, _TAIL - 1)
    tail_rows = jnp.take(tail_table, tail_idx, axis=0)
    return jnp.where((labels >= _SCAN_CLASSES)[:, None], tail_rows, out)


# speedup vs baseline: 4.2472x; 1.1144x over previous
import functools

import jax
import jax.numpy as jnp
from jax import lax
from jax.experimental import pallas as pl
from jax.experimental.pallas import tpu as pltpu
from jax.experimental.pallas import tpu_sc as plsc

NUM_CLASSES = 1000000
HIDDEN = 64
BATCH = 16384

_INFO = plsc.get_sparse_core_info()
_NC = _INFO.num_cores
_NS = _INFO.num_subcores
_NW = _NC * _NS
_LANES = _INFO.num_lanes

_TILE = 128
_NTC = (NUM_CLASSES + 1) // _TILE
_SCAN_CLASSES = _NTC * _TILE
_TAIL = NUM_CLASSES + 1 - _SCAN_CLASSES

_TC_PER_W = _NTC // _NW
_TC_EXTRA = _NTC - _TC_PER_W * _NW
_WIN = 384
_N_WIN_MAX = (_TC_PER_W * _TILE + _TILE + _WIN - 1) // _WIN

_FLUSH_CAP = 128
_FLUSH_HI = _FLUSH_CAP - _LANES
_BLK = 128
_STAGE_ROWS = BATCH + _FLUSH_CAP

_mesh = plsc.VectorSubcoreMesh(core_axis_name="c", subcore_axis_name="s")


def _popcount(mask):
    return plsc.all_reduce_population_count(mask)[0]


@functools.partial(
    pl.kernel,
    mesh=_mesh,
    out_type=jax.ShapeDtypeStruct((_STAGE_ROWS, _TILE), jnp.float32),
    scratch_types=[
        pltpu.VMEM((BATCH,), jnp.int32),
        pltpu.VMEM((BATCH,), jnp.int32),
        pltpu.VMEM((HIDDEN, _WIN), jnp.float32),
        pltpu.VMEM((HIDDEN, _WIN), jnp.float32),
        pltpu.VMEM((HIDDEN, _WIN), jnp.float32),
        pltpu.VMEM((_FLUSH_CAP, _TILE), jnp.float32),
        pltpu.VMEM((_FLUSH_CAP,), jnp.int32),
        pltpu.VMEM((_BLK * _LANES,), jnp.int32),
        pltpu.VMEM((_BLK * _LANES,), jnp.int32),
        pltpu.SemaphoreType.DMA,
        pltpu.SemaphoreType.DMA,
        pltpu.SemaphoreType.DMA,
        pltpu.SemaphoreType.DMA,
    ],
    compiler_params=pltpu.CompilerParams(
        use_tc_tiling_on_sc=True, needs_layout_passes=False
    ),
)
def _scan_kernel(labels_hbm, table_t_hbm, stage_hbm,
                 labels_v, pos_v, win_a, win_b, win_c, rows_v, rpos_v,
                 wrel_v, wpos_v, sem_a, sem_b, sem_c, sem):
    wid = lax.axis_index("s") * _NC + lax.axis_index("c")
    extra = jnp.minimum(wid, _TC_EXTRA)
    lo = (wid * _TC_PER_W + extra) * _TILE
    hi = lo + _TC_PER_W * _TILE + jnp.where(wid < _TC_EXTRA, _TILE, 0)
    n_win = (hi - lo + _WIN - 1) // _WIN

    lane = lax.iota(jnp.int32, _LANES)

    def compact(i, off):
        vec = labels_v[pl.ds(i * _LANES, _LANES)]
        m = (vec >= lo) & (vec < hi)
        plsc.store_compressed(pos_v.at[pl.ds(off, _LANES)], lane + i * _LANES, mask=m)
        return off + _popcount(m)

    def init_rpos(k, _):
        rpos_v[pl.ds(k * _LANES, _LANES)] = lane + (BATCH + k * _LANES)
        return ()

    lax.fori_loop(0, _FLUSH_CAP // _LANES, init_rpos, (), unroll=False)

    def flush(fcnt):
        copy = pltpu.make_async_copy(rows_v, stage_hbm.at[rpos_v], sem)
        copy.start()
        copy.wait()
        lax.fori_loop(0, _FLUSH_CAP // _LANES, init_rpos, (), unroll=False)
        del fcnt
        return 0

    def win_off(w):
        return jnp.minimum(lo + w * _WIN, hi - _WIN)

    def start(w, win_v, sem_w):
        pltpu.make_async_copy(
            table_t_hbm.at[:, pl.ds(win_off(w), _WIN)], win_v, sem_w
        ).start()

    def wait(win_v, sem_w):
        pltpu.make_async_copy(
            table_t_hbm.at[:, pl.ds(0, _WIN)], win_v, sem_w
        ).wait()

    def extract_window(w, win_v, fcnt):
        woff = win_off(w)

        def compress(j, wcnt):
            valid = (lane + j * _LANES) < mcount
            vpos = pos_v[pl.ds(j * _LANES, _LANES)]
            vcls = plsc.load_gather(labels_v, [vpos], mask=valid)
            inw = valid & (vcls >= woff) & (vcls < woff + _WIN) & (vcls < hi)
            plsc.store_compressed(
                wrel_v.at[pl.ds(wcnt, _LANES)], vcls - woff, mask=inw)
            plsc.store_compressed(
                wpos_v.at[pl.ds(wcnt, _LANES)], vpos, mask=inw)
            return wcnt + _popcount(inw)

        def dense(q, fcnt):
            valid = (lane + q * _LANES) < fcnt[1]
            rel = jnp.where(valid, wrel_v[pl.ds(q * _LANES, _LANES)], 0)
            vpos = wpos_v[pl.ds(q * _LANES, _LANES)]
            dest = fcnt[0] + lane
            for f in range(HIDDEN):
                vals = plsc.load_gather(
                    win_v, [jnp.full((_LANES,), f, jnp.int32), rel],
                    mask=valid)
                plsc.store_scatter(
                    rows_v, [dest, jnp.full((_LANES,), f, jnp.int32)],
                    vals, mask=valid)
            plsc.store_scatter(rpos_v, [dest], vpos, mask=valid)
            new = fcnt[0] + _popcount(valid)
            new = lax.cond(new >= _FLUSH_HI, flush, lambda c: c, new)
            return (new, fcnt[1])

        def block(b, fcnt):
            j0 = b * _BLK
            jn = jnp.minimum(mvregs - j0, _BLK)
            wcnt = lax.fori_loop(
                0, jn, lambda j, wc: compress(j0 + j, wc), 0, unroll=False)
            fcnt, _ = lax.fori_loop(
                0, (wcnt + _LANES - 1) // _LANES, dense, (fcnt, wcnt),
                unroll=False)
            return fcnt

        nblk = (mvregs + _BLK - 1) // _BLK
        return lax.fori_loop(0, nblk, block, fcnt, unroll=False)

    start(0, win_a, sem_a)
    start(1, win_b, sem_b)
    start(2, win_c, sem_c)
    labels_copy = pltpu.make_async_copy(labels_hbm, labels_v, sem)
    labels_copy.start()
    labels_copy.wait()
    mcount = lax.fori_loop(0, BATCH // _LANES, compact, 0, unroll=False)
    mvregs = (mcount + _LANES - 1) // _LANES

    def half(w, win_v, sem_w, fcnt):
        wait(win_v, sem_w)
        fcnt = extract_window(w, win_v, fcnt)
        return lax.cond(
            w + 3 < n_win,
            lambda c: (start(w + 3, win_v, sem_w), c)[1],
            lambda c: c,
            fcnt,
        )

    def guarded(w, win_v, sem_w, fcnt):
        return lax.cond(
            w < n_win,
            lambda c: half(w, win_v, sem_w, c),
            lambda c: c,
            fcnt,
        )

    def triple(t, fcnt):
        w0 = 3 * t
        fcnt = guarded(w0, win_a, sem_a, fcnt)
        fcnt = guarded(w0 + 1, win_b, sem_b, fcnt)
        return guarded(w0 + 2, win_c, sem_c, fcnt)

    fcnt = lax.fori_loop(0, (_N_WIN_MAX + 2) // 3, triple, 0, unroll=False)
    lax.cond(fcnt > 0, flush, lambda c: c, fcnt)


def kernel(labels, table):
    labels = labels.astype(jnp.int32)
    stage = _scan_kernel(labels, table.T)
    out = stage[:BATCH, :HIDDEN]
    tail_table = table[_SCAN_CLASSES:]
    tail_idx = jnp.clip(labels - _SCAN_CLASSES, 0, _TAIL - 1)
    tail_rows = jnp.take(tail_table, tail_idx, axis=0)
    return jnp.where((labels >= _SCAN_CLASSES)[:, None], tail_rows, out)
